# baseline (speedup 1.0000x reference)
import functools

import jax
import jax.numpy as jnp
from jax import lax
from jax.experimental import pallas as pl
from jax.experimental.pallas import tpu as pltpu
from jax.experimental.pallas import tpu_sc as plsc

VOCAB = 1000
D = 1000
DP = 1024
N_ROWS = 1024 * 200

_info = plsc.get_sparse_core_info()
NC, NS = _info.num_cores, _info.num_subcores
NW = NC * NS
ROWS_PER_W = N_ROWS // NW
CH = 64
N_CHUNKS = ROWS_PER_W // CH


@functools.partial(
    pl.kernel,
    mesh=plsc.VectorSubcoreMesh(core_axis_name="c", subcore_axis_name="s"),
    out_type=jax.ShapeDtypeStruct((N_ROWS, D), jnp.float32),
    scratch_types=[
        pltpu.VMEM((CH,), jnp.int32),
        pltpu.VMEM((CH, D), jnp.float32),
        pltpu.SemaphoreType.DMA,
    ],
    compiler_params=pltpu.CompilerParams(use_tc_tiling_on_sc=False),
)
def _gather_rows(idx_hbm, table_hbm, out_hbm, idx_v, rows_v, sem):
    wid = lax.axis_index("s") * NC + lax.axis_index("c")
    base = wid * ROWS_PER_W

    def body(g, carry):
        row0 = base + g * CH
        pltpu.sync_copy(idx_hbm.at[pl.ds(row0, CH)], idx_v)
        pltpu.async_copy(table_hbm.at[idx_v], rows_v, sem).wait()
        pltpu.sync_copy(rows_v, out_hbm.at[pl.ds(row0, CH)])
        return carry

    lax.fori_loop(0, N_CHUNKS, body, 0)


def kernel(idx, targets, table):
    flat = idx.reshape(N_ROWS)
    out = _gather_rows(flat, table)
    return out.reshape(idx.shape[0], idx.shape[1], D)


# speedup vs baseline: 1.0335x; 1.0335x over previous
import functools

import jax
import jax.numpy as jnp
from jax import lax
from jax.experimental import pallas as pl
from jax.experimental.pallas import tpu as pltpu
from jax.experimental.pallas import tpu_sc as plsc

VOCAB = 1000
D = 1000
N_ROWS = 1024 * 200

_info = plsc.get_sparse_core_info()
NC, NS = _info.num_cores, _info.num_subcores
NW = NC * NS
ROWS_PER_W = N_ROWS // NW
CH = 64
N_CHUNKS = ROWS_PER_W // CH
T = N_CHUNKS // 2


@functools.partial(
    pl.kernel,
    mesh=plsc.VectorSubcoreMesh(core_axis_name="c", subcore_axis_name="s"),
    out_type=jax.ShapeDtypeStruct((N_ROWS, D), jnp.float32),
    scratch_types=[
        pltpu.VMEM((CH,), jnp.int32),
        pltpu.VMEM((CH,), jnp.int32),
        pltpu.VMEM((CH, D), jnp.float32),
        pltpu.VMEM((CH, D), jnp.float32),
        pltpu.SemaphoreType.DMA,
        pltpu.SemaphoreType.DMA,
        pltpu.SemaphoreType.DMA,
        pltpu.SemaphoreType.DMA,
        pltpu.SemaphoreType.DMA,
        pltpu.SemaphoreType.DMA,
    ],
    compiler_params=pltpu.CompilerParams(use_tc_tiling_on_sc=False),
)
def _gather_rows(idx_hbm, table_hbm, out_hbm,
                 ib0, ib1, rb0, rb1, is0, is1, gs0, gs1, os0, os1):
    wid = lax.axis_index("s") * NC + lax.axis_index("c")
    base = wid * ROWS_PER_W
    ib = (ib0, ib1)
    rb = (rb0, rb1)
    isem = (is0, is1)
    gsem = (gs0, gs1)
    osem = (os0, os1)

    def idx_start(b, g):
        pltpu.async_copy(idx_hbm.at[pl.ds(base + g * CH, CH)], ib[b], isem[b])

    def idx_wait(b):
        pltpu.make_async_copy(idx_hbm.at[pl.ds(0, CH)], ib[b], isem[b]).wait()

    def gather_start(b):
        pltpu.async_copy(table_hbm.at[ib[b]], rb[b], gsem[b])

    def gather_wait(b):
        pltpu.make_async_copy(table_hbm.at[ib[b]], rb[b], gsem[b]).wait()

    def out_start(b, g):
        pltpu.async_copy(rb[b], out_hbm.at[pl.ds(base + g * CH, CH)], osem[b])

    def out_wait(b):
        pltpu.make_async_copy(rb[b], out_hbm.at[pl.ds(0, CH)], osem[b]).wait()

    idx_start(0, 0)
    idx_wait(0)
    gather_start(0)
    idx_start(1, 1)

    def body(t, carry):
        a = 2 * t

        gather_wait(0)
        idx_wait(1)

        @pl.when(t >= 1)
        def _():
            out_wait(1)

        gather_start(1)
        out_start(0, a)

        @pl.when(t < T - 1)
        def _():
            idx_start(0, a + 2)

        gather_wait(1)

        @pl.when(t < T - 1)
        def _():
            idx_wait(0)
            out_wait(0)
            gather_start(0)
            idx_start(1, a + 3)

        out_start(1, a + 1)
        return carry

    lax.fori_loop(0, T, body, 0)

    out_wait(0)
    out_wait(1)


def kernel(idx, targets, table):
    flat = idx.reshape(N_ROWS)
    out = _gather_rows(flat, table)
    return out.reshape(idx.shape[0], idx.shape[1], D)


# speedup vs baseline: 1.6878x; 1.6331x over previous
import functools

import jax
import jax.numpy as jnp
from jax import lax
from jax.experimental import pallas as pl
from jax.experimental.pallas import tpu as pltpu
from jax.experimental.pallas import tpu_sc as plsc

VOCAB = 1000
D = 1000
DM = 896
DT = 128
N_ROWS = 1024 * 200

_info = plsc.get_sparse_core_info()
NC, NS = _info.num_cores, _info.num_subcores
NW = NC * NS
ROWS_PER_W = N_ROWS // NW
CH = 40
N_CHUNKS = ROWS_PER_W // CH
T = N_CHUNKS // 2


@functools.partial(
    pl.kernel,
    mesh=plsc.VectorSubcoreMesh(core_axis_name="c", subcore_axis_name="s"),
    out_type=(
        jax.ShapeDtypeStruct((N_ROWS, D), jnp.float32),
        jax.ShapeDtypeStruct((N_ROWS, DT), jnp.float32),
    ),
    scratch_types=[
        pltpu.VMEM((CH,), jnp.int32),
        pltpu.VMEM((CH,), jnp.int32),
        pltpu.VMEM((CH, DM), jnp.float32),
        pltpu.VMEM((CH, DM), jnp.float32),
        pltpu.VMEM((CH, DT), jnp.float32),
        pltpu.VMEM((CH, DT), jnp.float32),
        pltpu.SemaphoreType.DMA,
        pltpu.SemaphoreType.DMA,
        pltpu.SemaphoreType.DMA,
        pltpu.SemaphoreType.DMA,
        pltpu.SemaphoreType.DMA,
        pltpu.SemaphoreType.DMA,
    ],
)
def _gather_rows(idx_hbm, tmain_hbm, ttail_hbm, out_hbm, tail_hbm,
                 ib0, ib1, mb0, mb1, tb0, tb1, is0, is1, gs0, gs1, os0, os1):
    wid = lax.axis_index("s") * NC + lax.axis_index("c")
    base = wid * ROWS_PER_W
    ib = (ib0, ib1)
    mb = (mb0, mb1)
    tb = (tb0, tb1)
    isem = (is0, is1)
    gsem = (gs0, gs1)
    osem = (os0, os1)

    def idx_start(b, g):
        pltpu.async_copy(idx_hbm.at[pl.ds(base + g * CH, CH)], ib[b], isem[b])

    def idx_wait(b):
        pltpu.make_async_copy(idx_hbm.at[pl.ds(0, CH)], ib[b], isem[b]).wait()

    def gather_start(b):
        pltpu.async_copy(tmain_hbm.at[ib[b]], mb[b], gsem[b])
        pltpu.async_copy(ttail_hbm.at[ib[b]], tb[b], gsem[b])

    def gather_wait(b):
        pltpu.make_async_copy(tmain_hbm.at[ib[b]], mb[b], gsem[b]).wait()
        pltpu.make_async_copy(ttail_hbm.at[ib[b]], tb[b], gsem[b]).wait()

    def out_start(b, g):
        r = base + g * CH
        pltpu.async_copy(mb[b], out_hbm.at[pl.ds(r, CH), pl.ds(0, DM)], osem[b])
        pltpu.async_copy(tb[b], tail_hbm.at[pl.ds(r, CH)], osem[b])

    def out_wait(b):
        pltpu.make_async_copy(mb[b], out_hbm.at[pl.ds(0, CH), pl.ds(0, DM)],
                              osem[b]).wait()
        pltpu.make_async_copy(tb[b], tail_hbm.at[pl.ds(0, CH)], osem[b]).wait()

    idx_start(0, 0)
    idx_wait(0)
    gather_start(0)
    idx_start(1, 1)

    def body(t, carry):
        a = 2 * t

        gather_wait(0)
        idx_wait(1)

        @pl.when(t >= 1)
        def _():
            out_wait(1)

        gather_start(1)
        out_start(0, a)

        @pl.when(t < T - 1)
        def _():
            idx_start(0, a + 2)

        gather_wait(1)

        @pl.when(t < T - 1)
        def _():
            idx_wait(0)
            out_wait(0)
            gather_start(0)
            idx_start(1, a + 3)

        out_start(1, a + 1)
        return carry

    lax.fori_loop(0, T, body, 0)

    out_wait(0)
    out_wait(1)


def kernel(idx, targets, table):
    flat = idx.reshape(N_ROWS)
    table_main = table[:, :DM]
    table_tail = jnp.pad(table[:, DM:], ((0, 0), (0, DT - (D - DM))))
    out, tail = _gather_rows(flat, table_main, table_tail)
    out = lax.dynamic_update_slice(out, tail[:, : D - DM], (0, DM))
    return out.reshape(idx.shape[0], idx.shape[1], D)
